# Initial kernel scaffold; baseline (speedup 1.0000x reference)
import jax
import jax.numpy as jnp
from jax.experimental import pallas as pl


def kernel(x, edge_index, W1, b1, W2, b2):
    raise NotImplementedError("write your pallas kernel here")


# speedup vs baseline: 2.4837x; 2.4837x over previous
import functools

import jax
import jax.numpy as jnp
from jax import lax
from jax.experimental import pallas as pl
from jax.experimental.pallas import tpu as pltpu
from jax.experimental.pallas import tpu_sc as plsc

N = 10000
NP = 10240
D = 64
K = 10
ALPHA = 0.1
NC, NS, L = 2, 16, 16
NW = NC * NS
RPT = NP // NW
ACCR = RPT + 8
CHUNK = 128
CAP = 24576
SCAN = 2000
CAPP = CAP + CHUNK

_mesh = plsc.VectorSubcoreMesh(
    core_axis_name="c", subcore_axis_name="s", num_cores=NC, num_subcores=NS)


def _wid():
  return lax.axis_index("c") * NS + lax.axis_index("s")


@functools.partial(
    pl.kernel,
    out_type=[
        jax.ShapeDtypeStruct((NW, CAPP), jnp.int32),
        jax.ShapeDtypeStruct((NW, CAPP), jnp.int32),
        jax.ShapeDtypeStruct((NW, 16), jnp.int32),
        jax.ShapeDtypeStruct((NP, D), jnp.float32),
    ],
    mesh=_mesh,
    compiler_params=pltpu.CompilerParams(needs_layout_passes=False, use_tc_tiling_on_sc=False),
    scratch_types=[
        pltpu.VMEM((SCAN,), jnp.int32),
        pltpu.VMEM((SCAN,), jnp.int32),
        pltpu.VMEM((CAPP,), jnp.int32),
        pltpu.VMEM((CAPP,), jnp.int32),
        pltpu.VMEM((RPT,), jnp.float32),
        pltpu.VMEM((RPT, D), jnp.float32),
        pltpu.VMEM((16,), jnp.int32),
        pltpu.SemaphoreType.DMA,
    ],
)
def _pass_a(src_hbm, dst_hbm, srcl_hbm, dstl_hbm, cnt_hbm, disb_hbm,
            srcv, dstv, srcl, dstl, degv, disbv, cntv, sem):
  del sem
  wid = _wid()
  lo = wid * RPT
  E = src_hbm.shape[0]

  def zero_deg(g, _):
    degv[pl.ds(g * L, L)] = jnp.zeros((L,), jnp.float32)
    return 0
  lax.fori_loop(0, RPT // L, zero_deg, 0)

  ones = jnp.ones((L,), jnp.float32)

  def scan_chunk(i, c):
    pltpu.sync_copy(src_hbm.at[pl.ds(i * SCAN, SCAN)], srcv)
    pltpu.sync_copy(dst_hbm.at[pl.ds(i * SCAN, SCAN)], dstv)

    def grp(j, c):
      s = srcv[pl.ds(j * L, L)]
      dl = dstv[pl.ds(j * L, L)] - lo
      m = (dl >= 0) & (dl < RPT) & (c < CAP - L)
      csum = plsc.cumsum(jnp.where(m, 1, 0))
      pos = c + csum - 1
      plsc.store_scatter(srcl, [pos], s, mask=m)
      plsc.store_scatter(dstl, [pos], dl, mask=m)
      plsc.addupdate_scatter(degv, [dl], ones, mask=m)
      return c + jnp.max(csum)

    return lax.fori_loop(0, SCAN // L, grp, c)

  c = lax.fori_loop(0, E // SCAN, scan_chunk, jnp.int32(0))

  def fill(i, _):
    off = c + i * L
    srcl[pl.ds(off, L)] = jnp.zeros((L,), jnp.int32)
    dstl[pl.ds(off, L)] = jnp.full((L,), RPT, jnp.int32)
    return 0
  lax.fori_loop(0, CHUNK // L, fill, 0)
  cpad = ((c + CHUNK - 1) // CHUNK) * CHUNK

  pltpu.sync_copy(srcl, srcl_hbm.at[wid])
  pltpu.sync_copy(dstl, dstl_hbm.at[wid])
  cntv[...] = jnp.full((16,), cpad // CHUNK, jnp.int32)
  pltpu.sync_copy(cntv, cnt_hbm.at[wid])

  def newton(g, _):
    x = degv[pl.ds(g * L, L)] + 1.0
    yi = jnp.int32(0x5F3759DF) - (plsc.bitcast(x, jnp.int32) >> 1)
    y = plsc.bitcast(yi, jnp.float32)
    for _ in range(3):
      y = y * (1.5 - 0.5 * x * y * y)
    degv[pl.ds(g * L, L)] = y
    return 0
  lax.fori_loop(0, RPT // L, newton, 0)

  def bcast_row(r, _):
    splat = plsc.load_gather(degv, [jnp.full((L,), r, jnp.int32)])
    for f in range(D // L):
      disbv[r, pl.ds(f * L, L)] = splat
    return 0
  lax.fori_loop(0, RPT, bcast_row, 0)
  pltpu.sync_copy(disbv, disb_hbm.at[pl.ds(lo, RPT)])


@functools.partial(
    pl.kernel,
    out_type=[
        jax.ShapeDtypeStruct((NP, D), jnp.float32),
        jax.ShapeDtypeStruct((NP, D), jnp.float32),
    ],
    mesh=_mesh,
    compiler_params=pltpu.CompilerParams(needs_layout_passes=False, use_tc_tiling_on_sc=False),
    scratch_types=[
        pltpu.VMEM((ACCR, D), jnp.float32),
        pltpu.VMEM((1, CHUNK), jnp.int32),
        pltpu.VMEM((1, CHUNK), jnp.int32),
        pltpu.VMEM((CHUNK, D), jnp.float32),
        pltpu.VMEM((16,), jnp.int32),
        pltpu.VMEM((RPT, D), jnp.float32),
        pltpu.VMEM((RPT, D), jnp.float32),
        pltpu.VMEM((RPT, D), jnp.float32),
        pltpu.SemaphoreType.DMA,
    ],
)
def _step(outp_hbm, h_hbm, disb_hbm, srcl_hbm, dstl_hbm, cnt_hbm,
          outp_new_hbm, out_new_hbm,
          acc, sidx, didx, buf, cntv, hv, opv, dbv, sem):
  wid = _wid()
  lo = wid * RPT
  zero = jnp.zeros((L,), jnp.float32)

  def zero_acc(r, _):
    for f in range(D // L):
      acc[r, pl.ds(f * L, L)] = zero
    return 0
  lax.fori_loop(0, ACCR, zero_acc, 0)

  pltpu.sync_copy(cnt_hbm.at[wid], cntv)
  nch = jnp.max(cntv[...])
  lanes = lax.iota(jnp.int32, L)

  def chunk(i, _):
    pltpu.sync_copy(srcl_hbm.at[wid, pl.ds(i * CHUNK, CHUNK)], sidx.at[0])
    pltpu.sync_copy(dstl_hbm.at[wid, pl.ds(i * CHUNK, CHUNK)], didx.at[0])
    pltpu.async_copy(outp_hbm.at[sidx.at[0]], buf, sem).wait()

    def grp(g, _):
      e = g * L + lanes
      dl = didx[0, pl.ds(g * L, L)]
      for f in range(D):
        col = jnp.full((L,), f, jnp.int32)
        val = plsc.load_gather(buf, [e, col])
        plsc.addupdate_scatter(acc, [dl, col], val)
      return 0
    lax.fori_loop(0, CHUNK // L, grp, 0)
    return 0
  lax.fori_loop(0, nch, chunk, 0)

  pltpu.sync_copy(h_hbm.at[pl.ds(lo, RPT)], hv)
  pltpu.sync_copy(outp_hbm.at[pl.ds(lo, RPT)], opv)
  pltpu.sync_copy(disb_hbm.at[pl.ds(lo, RPT)], dbv)

  def node(r, _):
    for f in range(D // L):
      sl = (r, pl.ds(f * L, L))
      t = ((1.0 - ALPHA) * dbv[sl] * (acc[sl] + opv[sl])
           + ALPHA * hv[sl])
      opv[sl] = dbv[sl] * t
      hv[sl] = t
    return 0
  lax.fori_loop(0, RPT, node, 0)

  pltpu.sync_copy(opv, outp_new_hbm.at[pl.ds(lo, RPT)])
  pltpu.sync_copy(hv, out_new_hbm.at[pl.ds(lo, RPT)])


_MLP_BLK = 1024


def _mlp_body(x_ref, w1_ref, b1_ref, w2_ref, b2_ref, disb_ref, h_ref, op_ref):
  x = x_ref[...]
  h1 = lax.dot_general(x, w1_ref[...], (((1,), (1,)), ((), ())),
                       preferred_element_type=jnp.float32) + b1_ref[...]
  h1 = jnp.maximum(h1, 0.0)
  h = lax.dot_general(h1, w2_ref[...], (((1,), (1,)), ((), ())),
                      preferred_element_type=jnp.float32) + b2_ref[...]
  h_ref[...] = h
  op_ref[...] = disb_ref[...] * h


def _mlp(xp, W1, b1, W2, b2, disb):
  grid = (NP // _MLP_BLK,)
  return pl.pallas_call(
      _mlp_body,
      grid=grid,
      in_specs=[
          pl.BlockSpec((_MLP_BLK, 128), lambda i: (i, 0)),
          pl.BlockSpec((256, 128), lambda i: (0, 0)),
          pl.BlockSpec((1, 256), lambda i: (0, 0)),
          pl.BlockSpec((64, 256), lambda i: (0, 0)),
          pl.BlockSpec((1, 64), lambda i: (0, 0)),
          pl.BlockSpec((_MLP_BLK, D), lambda i: (i, 0)),
      ],
      out_specs=[
          pl.BlockSpec((_MLP_BLK, D), lambda i: (i, 0)),
          pl.BlockSpec((_MLP_BLK, D), lambda i: (i, 0)),
      ],
      out_shape=[
          jax.ShapeDtypeStruct((NP, D), jnp.float32),
          jax.ShapeDtypeStruct((NP, D), jnp.float32),
      ],
  )(xp, W1, b1, W2, b2, disb)


def kernel(x, edge_index, W1, b1, W2, b2):
  src = edge_index[0].astype(jnp.int32)
  dst = edge_index[1].astype(jnp.int32)
  xp = jnp.pad(x, ((0, NP - N), (0, 0)))
  srcl, dstl, cnt, disb = _pass_a(src, dst)
  h, outp = _mlp(xp, W1, b1.reshape(1, -1), W2, b2.reshape(1, -1), disb)
  out = h
  for _ in range(K):
    outp, out = _step(outp, h, disb, srcl, dstl, cnt)
  return out[:N]


# speedup vs baseline: 2.8347x; 1.1413x over previous
import functools

import jax
import jax.numpy as jnp
from jax import lax
from jax.experimental import pallas as pl
from jax.experimental.pallas import tpu as pltpu
from jax.experimental.pallas import tpu_sc as plsc

N = 10000
NP = 10240
D = 64
K = 10
ALPHA = 0.1
NC, NS, L = 2, 16, 16
NW = NC * NS
RPT = NP // NW
ACCR = RPT + 8
CHUNK = 128
CAP = 24576
SCAN = 2000
CAPP = CAP + CHUNK

_mesh = plsc.VectorSubcoreMesh(
    core_axis_name="c", subcore_axis_name="s", num_cores=NC, num_subcores=NS)


def _wid():
  return lax.axis_index("c") * NS + lax.axis_index("s")


@functools.partial(
    pl.kernel,
    out_type=[
        jax.ShapeDtypeStruct((NW, CAPP), jnp.int32),
        jax.ShapeDtypeStruct((NW, CAPP), jnp.int32),
        jax.ShapeDtypeStruct((NW, 16), jnp.int32),
        jax.ShapeDtypeStruct((NP, D), jnp.float32),
    ],
    mesh=_mesh,
    compiler_params=pltpu.CompilerParams(needs_layout_passes=False, use_tc_tiling_on_sc=False),
    scratch_types=[
        pltpu.VMEM((SCAN,), jnp.int32),
        pltpu.VMEM((SCAN,), jnp.int32),
        pltpu.VMEM((CAPP,), jnp.int32),
        pltpu.VMEM((CAPP,), jnp.int32),
        pltpu.VMEM((RPT,), jnp.float32),
        pltpu.VMEM((RPT, D), jnp.float32),
        pltpu.VMEM((16,), jnp.int32),
        pltpu.SemaphoreType.DMA,
    ],
)
def _pass_a(src_hbm, dst_hbm, srcl_hbm, dstl_hbm, cnt_hbm, disb_hbm,
            srcv, dstv, srcl, dstl, degv, disbv, cntv, sem):
  del sem
  wid = _wid()
  lo = wid * RPT
  E = src_hbm.shape[0]

  def zero_deg(g, _):
    degv[pl.ds(g * L, L)] = jnp.zeros((L,), jnp.float32)
    return 0
  lax.fori_loop(0, RPT // L, zero_deg, 0)

  ones = jnp.ones((L,), jnp.float32)

  def scan_chunk(i, c):
    pltpu.sync_copy(src_hbm.at[pl.ds(i * SCAN, SCAN)], srcv)
    pltpu.sync_copy(dst_hbm.at[pl.ds(i * SCAN, SCAN)], dstv)

    def grp(j, c):
      s = srcv[pl.ds(j * L, L)]
      dl = dstv[pl.ds(j * L, L)] - lo
      m = (dl >= 0) & (dl < RPT) & (c < CAP - L)
      csum = plsc.cumsum(jnp.where(m, 1, 0))
      pos = c + csum - 1
      plsc.store_scatter(srcl, [pos], s, mask=m)
      plsc.store_scatter(dstl, [pos], dl, mask=m)
      plsc.addupdate_scatter(degv, [dl], ones, mask=m)
      return c + jnp.max(csum)

    return lax.fori_loop(0, SCAN // L, grp, c)

  c = lax.fori_loop(0, E // SCAN, scan_chunk, jnp.int32(0))

  def fill(i, _):
    off = c + i * L
    srcl[pl.ds(off, L)] = jnp.zeros((L,), jnp.int32)
    dstl[pl.ds(off, L)] = jnp.full((L,), RPT, jnp.int32)
    return 0
  lax.fori_loop(0, CHUNK // L, fill, 0)
  cpad = ((c + CHUNK - 1) // CHUNK) * CHUNK

  pltpu.sync_copy(srcl, srcl_hbm.at[wid])
  pltpu.sync_copy(dstl, dstl_hbm.at[wid])
  cntv[...] = jnp.full((16,), cpad // CHUNK, jnp.int32)
  pltpu.sync_copy(cntv, cnt_hbm.at[wid])

  def newton(g, _):
    x = degv[pl.ds(g * L, L)] + 1.0
    yi = jnp.int32(0x5F3759DF) - (plsc.bitcast(x, jnp.int32) >> 1)
    y = plsc.bitcast(yi, jnp.float32)
    for _ in range(3):
      y = y * (1.5 - 0.5 * x * y * y)
    degv[pl.ds(g * L, L)] = y
    return 0
  lax.fori_loop(0, RPT // L, newton, 0)

  def bcast_row(r, _):
    splat = plsc.load_gather(degv, [jnp.full((L,), r, jnp.int32)])
    for f in range(D // L):
      disbv[r, pl.ds(f * L, L)] = splat
    return 0
  lax.fori_loop(0, RPT, bcast_row, 0)
  pltpu.sync_copy(disbv, disb_hbm.at[pl.ds(lo, RPT)])


NSUB = 4
NB = 64


@functools.partial(
    pl.kernel,
    out_type=[
        jax.ShapeDtypeStruct((NP, D), jnp.float32),
        jax.ShapeDtypeStruct((NP, D), jnp.float32),
    ],
    mesh=_mesh,
    compiler_params=pltpu.CompilerParams(needs_layout_passes=False, use_tc_tiling_on_sc=False),
    scratch_types=[
        pltpu.VMEM((ACCR, D), jnp.float32),
        pltpu.VMEM((CAP,), jnp.int32),
        pltpu.VMEM((CAP,), jnp.int32),
        pltpu.VMEM((NSUB, CHUNK, D), jnp.float32),
        pltpu.VMEM((16,), jnp.int32),
        pltpu.VMEM((NB, D), jnp.float32),
        pltpu.VMEM((NB, D), jnp.float32),
        pltpu.VMEM((NB, D), jnp.float32),
        pltpu.SemaphoreType.DMA,
        pltpu.SemaphoreType.DMA,
        pltpu.SemaphoreType.DMA,
        pltpu.SemaphoreType.DMA,
    ],
)
def _step(outp_hbm, h_hbm, disb_hbm, srcl_hbm, dstl_hbm, cnt_hbm,
          outp_new_hbm, out_new_hbm,
          acc, sidxa, didxa, rowb, cntv, hb, ob, db,
          sem0, sem1, sem2, sem3):
  wid = _wid()
  lo = wid * RPT
  zero = jnp.zeros((L,), jnp.float32)
  sems = (sem0, sem1, sem2, sem3)

  def zero_acc(r, _):
    for f in range(D // L):
      acc[r, pl.ds(f * L, L)] = zero
    return 0
  lax.fori_loop(0, ACCR, zero_acc, 0)

  pltpu.sync_copy(cnt_hbm.at[wid], cntv)
  nch = jnp.max(cntv[...])
  pltpu.sync_copy(srcl_hbm.at[wid, pl.ds(0, CAP)], sidxa)
  pltpu.sync_copy(dstl_hbm.at[wid, pl.ds(0, CAP)], didxa)
  lanes = lax.iota(jnp.int32, L)

  def fire(i, slot):
    pltpu.async_copy(outp_hbm.at[sidxa.at[pl.ds(i * CHUNK, CHUNK)]],
                     rowb.at[slot], sems[slot])

  def wait_slot(slot):
    pltpu.make_async_copy(outp_hbm.at[pl.ds(0, CHUNK)], rowb.at[slot],
                          sems[slot]).wait()

  for j in range(NSUB - 1):
    @pl.when(j < nch)
    def _():
      fire(j, j)

  def blk(b, _):
    for j in range(NSUB):
      i = b * NSUB + j

      @pl.when(i < nch)
      def _():
        @pl.when(i + NSUB - 1 < nch)
        def _():
          fire(i + NSUB - 1, (j + NSUB - 1) % NSUB)
        wait_slot(j)
        base = i * CHUNK

        def grp(g, _):
          e = g * L + lanes
          dl = didxa[pl.ds(base + g * L, L)]
          for f in range(D):
            col = jnp.full((L,), f, jnp.int32)
            val = plsc.load_gather(rowb.at[j], [e, col])
            plsc.addupdate_scatter(acc, [dl, col], val)
          return 0
        lax.fori_loop(0, CHUNK // L, grp, 0)
    return 0
  lax.fori_loop(0, (nch + NSUB - 1) // NSUB, blk, 0)

  def nodeblk(nb, _):
    r0 = lo + nb * NB
    a0 = nb * NB
    pltpu.sync_copy(h_hbm.at[pl.ds(r0, NB)], hb)
    pltpu.sync_copy(outp_hbm.at[pl.ds(r0, NB)], ob)
    pltpu.sync_copy(disb_hbm.at[pl.ds(r0, NB)], db)

    def row(r, _):
      for f in range(D // L):
        sl = (r, pl.ds(f * L, L))
        t = ((1.0 - ALPHA) * db[sl] * (acc[a0 + r, pl.ds(f * L, L)] + ob[sl])
             + ALPHA * hb[sl])
        hb[sl] = t
        ob[sl] = db[sl] * t
      return 0
    lax.fori_loop(0, NB, row, 0)
    pltpu.sync_copy(ob, outp_new_hbm.at[pl.ds(r0, NB)])
    pltpu.sync_copy(hb, out_new_hbm.at[pl.ds(r0, NB)])
    return 0
  lax.fori_loop(0, RPT // NB, nodeblk, 0)


_MLP_BLK = 1024


def _mlp_body(x_ref, w1_ref, b1_ref, w2_ref, b2_ref, disb_ref, h_ref, op_ref):
  x = x_ref[...]
  h1 = lax.dot_general(x, w1_ref[...], (((1,), (1,)), ((), ())),
                       preferred_element_type=jnp.float32) + b1_ref[...]
  h1 = jnp.maximum(h1, 0.0)
  h = lax.dot_general(h1, w2_ref[...], (((1,), (1,)), ((), ())),
                      preferred_element_type=jnp.float32) + b2_ref[...]
  h_ref[...] = h
  op_ref[...] = disb_ref[...] * h


def _mlp(xp, W1, b1, W2, b2, disb):
  grid = (NP // _MLP_BLK,)
  return pl.pallas_call(
      _mlp_body,
      grid=grid,
      in_specs=[
          pl.BlockSpec((_MLP_BLK, 128), lambda i: (i, 0)),
          pl.BlockSpec((256, 128), lambda i: (0, 0)),
          pl.BlockSpec((1, 256), lambda i: (0, 0)),
          pl.BlockSpec((64, 256), lambda i: (0, 0)),
          pl.BlockSpec((1, 64), lambda i: (0, 0)),
          pl.BlockSpec((_MLP_BLK, D), lambda i: (i, 0)),
      ],
      out_specs=[
          pl.BlockSpec((_MLP_BLK, D), lambda i: (i, 0)),
          pl.BlockSpec((_MLP_BLK, D), lambda i: (i, 0)),
      ],
      out_shape=[
          jax.ShapeDtypeStruct((NP, D), jnp.float32),
          jax.ShapeDtypeStruct((NP, D), jnp.float32),
      ],
  )(xp, W1, b1, W2, b2, disb)


def kernel(x, edge_index, W1, b1, W2, b2):
  src = edge_index[0].astype(jnp.int32)
  dst = edge_index[1].astype(jnp.int32)
  xp = jnp.pad(x, ((0, NP - N), (0, 0)))
  srcl, dstl, cnt, disb = _pass_a(src, dst)
  h, outp = _mlp(xp, W1, b1.reshape(1, -1), W2, b2.reshape(1, -1), disb)
  out = h
  for _ in range(K):
    outp, out = _step(outp, h, disb, srcl, dstl, cnt)
  return out[:N]


# speedup vs baseline: 12.7537x; 4.4992x over previous
import functools

import jax
import jax.numpy as jnp
from jax import lax
from jax.experimental import pallas as pl
from jax.experimental.pallas import tpu as pltpu
from jax.experimental.pallas import tpu_sc as plsc

N = 10000
NP = 10240
D = 64
K = 10
ALPHA = 0.1
NC, NS, L = 2, 16, 16
NW = NC * NS
RPT = NP // NW
ACCR = RPT + 8
CHUNK = 128
CAP = 24576
SCAN = 2000
CAPP = CAP + CHUNK

_mesh = plsc.VectorSubcoreMesh(
    core_axis_name="c", subcore_axis_name="s", num_cores=NC, num_subcores=NS)


def _wid():
  return lax.axis_index("c") * NS + lax.axis_index("s")


@functools.partial(
    pl.kernel,
    out_type=[
        jax.ShapeDtypeStruct((NW, CAPP), jnp.int32),
        jax.ShapeDtypeStruct((NW, CAPP), jnp.int32),
        jax.ShapeDtypeStruct((NW, 16), jnp.int32),
        jax.ShapeDtypeStruct((NP, D), jnp.float32),
    ],
    mesh=_mesh,
    compiler_params=pltpu.CompilerParams(needs_layout_passes=False, use_tc_tiling_on_sc=False),
    scratch_types=[
        pltpu.VMEM((SCAN,), jnp.int32),
        pltpu.VMEM((SCAN,), jnp.int32),
        pltpu.VMEM((CAPP,), jnp.int32),
        pltpu.VMEM((CAPP,), jnp.int32),
        pltpu.VMEM((RPT,), jnp.float32),
        pltpu.VMEM((RPT, D), jnp.float32),
        pltpu.VMEM((16,), jnp.int32),
        pltpu.SemaphoreType.DMA,
    ],
)
def _pass_a(src_hbm, dst_hbm, srcl_hbm, dstl_hbm, cnt_hbm, disb_hbm,
            srcv, dstv, srcl, dstl, degv, disbv, cntv, sem):
  del sem
  wid = _wid()
  lo = wid * RPT
  E = src_hbm.shape[0]

  def zero_deg(g, _):
    degv[pl.ds(g * L, L)] = jnp.zeros((L,), jnp.float32)
    return 0
  lax.fori_loop(0, RPT // L, zero_deg, 0)

  ones = jnp.ones((L,), jnp.float32)

  def scan_chunk(i, c):
    pltpu.sync_copy(src_hbm.at[pl.ds(i * SCAN, SCAN)], srcv)
    pltpu.sync_copy(dst_hbm.at[pl.ds(i * SCAN, SCAN)], dstv)

    def grp(j, c):
      s = srcv[pl.ds(j * L, L)]
      dl = dstv[pl.ds(j * L, L)] - lo
      m = (dl >= 0) & (dl < RPT) & (c < CAP - L)
      csum = plsc.cumsum(jnp.where(m, 1, 0))
      pos = c + csum - 1
      plsc.store_scatter(srcl, [pos], s, mask=m)
      plsc.store_scatter(dstl, [pos], dl, mask=m)
      plsc.addupdate_scatter(degv, [dl], ones, mask=m)
      return c + jnp.max(csum)

    return lax.fori_loop(0, SCAN // L, grp, c)

  c = lax.fori_loop(0, E // SCAN, scan_chunk, jnp.int32(0))

  def fill(i, _):
    off = c + i * L
    srcl[pl.ds(off, L)] = jnp.zeros((L,), jnp.int32)
    dstl[pl.ds(off, L)] = jnp.full((L,), RPT, jnp.int32)
    return 0
  lax.fori_loop(0, CHUNK // L, fill, 0)
  cpad = ((c + CHUNK - 1) // CHUNK) * CHUNK

  pltpu.sync_copy(srcl, srcl_hbm.at[wid])
  pltpu.sync_copy(dstl, dstl_hbm.at[wid])
  cntv[...] = jnp.full((16,), cpad // CHUNK, jnp.int32)
  pltpu.sync_copy(cntv, cnt_hbm.at[wid])

  def newton(g, _):
    x = degv[pl.ds(g * L, L)] + 1.0
    yi = jnp.int32(0x5F3759DF) - (plsc.bitcast(x, jnp.int32) >> 1)
    y = plsc.bitcast(yi, jnp.float32)
    for _ in range(3):
      y = y * (1.5 - 0.5 * x * y * y)
    degv[pl.ds(g * L, L)] = y
    return 0
  lax.fori_loop(0, RPT // L, newton, 0)

  def bcast_row(r, _):
    splat = plsc.load_gather(degv, [jnp.full((L,), r, jnp.int32)])
    for f in range(D // L):
      disbv[r, pl.ds(f * L, L)] = splat
    return 0
  lax.fori_loop(0, RPT, bcast_row, 0)
  pltpu.sync_copy(disbv, disb_hbm.at[pl.ds(lo, RPT)])


NSUB = 4
NB = 64


@functools.partial(
    pl.kernel,
    out_type=[
        jax.ShapeDtypeStruct((NP, D), jnp.float32),
        jax.ShapeDtypeStruct((NP, D), jnp.float32),
    ],
    mesh=_mesh,
    compiler_params=pltpu.CompilerParams(needs_layout_passes=False, use_tc_tiling_on_sc=False),
    scratch_types=[
        pltpu.VMEM((ACCR, D), jnp.float32),
        pltpu.VMEM((CAP,), jnp.int32),
        pltpu.VMEM((CAP,), jnp.int32),
        pltpu.VMEM((NSUB, CHUNK, D), jnp.float32),
        pltpu.VMEM((16,), jnp.int32),
        pltpu.VMEM((NB, D), jnp.float32),
        pltpu.VMEM((NB, D), jnp.float32),
        pltpu.VMEM((NB, D), jnp.float32),
        pltpu.SemaphoreType.DMA,
        pltpu.SemaphoreType.DMA,
        pltpu.SemaphoreType.DMA,
        pltpu.SemaphoreType.DMA,
    ],
)
def _step(outp_hbm, h_hbm, disb_hbm, srcl_hbm, dstl_hbm, cnt_hbm,
          outp_new_hbm, out_new_hbm,
          acc, sidxa, didxa, rowb, cntv, hb, ob, db,
          sem0, sem1, sem2, sem3):
  wid = _wid()
  lo = wid * RPT
  zero = jnp.zeros((L,), jnp.float32)
  sems = (sem0, sem1, sem2, sem3)

  def zero_acc(r, _):
    for f in range(D // L):
      acc[r, pl.ds(f * L, L)] = zero
    return 0
  lax.fori_loop(0, ACCR, zero_acc, 0)

  pltpu.sync_copy(cnt_hbm.at[wid], cntv)
  nch = jnp.max(cntv[...])
  pltpu.sync_copy(srcl_hbm.at[wid, pl.ds(0, CAP)], sidxa)
  pltpu.sync_copy(dstl_hbm.at[wid, pl.ds(0, CAP)], didxa)
  lanes = lax.iota(jnp.int32, L)

  def fire(i, slot):
    pltpu.async_copy(outp_hbm.at[sidxa.at[pl.ds(i * CHUNK, CHUNK)]],
                     rowb.at[slot], sems[slot])

  def wait_slot(slot):
    pltpu.make_async_copy(outp_hbm.at[pl.ds(0, CHUNK)], rowb.at[slot],
                          sems[slot]).wait()

  for j in range(NSUB - 1):
    @pl.when(j < nch)
    def _():
      fire(j, j)

  def blk(b, _):
    for j in range(NSUB):
      i = b * NSUB + j

      @pl.when(i < nch)
      def _():
        @pl.when(i + NSUB - 1 < nch)
        def _():
          fire(i + NSUB - 1, (j + NSUB - 1) % NSUB)
        wait_slot(j)
        base = i * CHUNK

        def grp(g, _):
          dl = didxa[pl.ds(base + g * L, L)]
          for jj in range(L):
            dlj = dl[jj]
            vals = [rowb[j, g * L + jj, pl.ds(f * L, L)]
                    for f in range(D // L)]
            for f in range(D // L):
              plsc.addupdate(acc.at[dlj, pl.ds(f * L, L)], vals[f])
          return 0
        lax.fori_loop(0, CHUNK // L, grp, 0)
    return 0
  lax.fori_loop(0, (nch + NSUB - 1) // NSUB, blk, 0)

  def nodeblk(nb, _):
    r0 = lo + nb * NB
    a0 = nb * NB
    pltpu.sync_copy(h_hbm.at[pl.ds(r0, NB)], hb)
    pltpu.sync_copy(outp_hbm.at[pl.ds(r0, NB)], ob)
    pltpu.sync_copy(disb_hbm.at[pl.ds(r0, NB)], db)

    def row(r, _):
      for f in range(D // L):
        sl = (r, pl.ds(f * L, L))
        t = ((1.0 - ALPHA) * db[sl] * (acc[a0 + r, pl.ds(f * L, L)] + ob[sl])
             + ALPHA * hb[sl])
        hb[sl] = t
        ob[sl] = db[sl] * t
      return 0
    lax.fori_loop(0, NB, row, 0)
    pltpu.sync_copy(ob, outp_new_hbm.at[pl.ds(r0, NB)])
    pltpu.sync_copy(hb, out_new_hbm.at[pl.ds(r0, NB)])
    return 0
  lax.fori_loop(0, RPT // NB, nodeblk, 0)


_MLP_BLK = 1024


def _mlp_body(x_ref, w1_ref, b1_ref, w2_ref, b2_ref, disb_ref, h_ref, op_ref):
  x = x_ref[...]
  h1 = lax.dot_general(x, w1_ref[...], (((1,), (1,)), ((), ())),
                       preferred_element_type=jnp.float32) + b1_ref[...]
  h1 = jnp.maximum(h1, 0.0)
  h = lax.dot_general(h1, w2_ref[...], (((1,), (1,)), ((), ())),
                      preferred_element_type=jnp.float32) + b2_ref[...]
  h_ref[...] = h
  op_ref[...] = disb_ref[...] * h


def _mlp(xp, W1, b1, W2, b2, disb):
  grid = (NP // _MLP_BLK,)
  return pl.pallas_call(
      _mlp_body,
      grid=grid,
      in_specs=[
          pl.BlockSpec((_MLP_BLK, 128), lambda i: (i, 0)),
          pl.BlockSpec((256, 128), lambda i: (0, 0)),
          pl.BlockSpec((1, 256), lambda i: (0, 0)),
          pl.BlockSpec((64, 256), lambda i: (0, 0)),
          pl.BlockSpec((1, 64), lambda i: (0, 0)),
          pl.BlockSpec((_MLP_BLK, D), lambda i: (i, 0)),
      ],
      out_specs=[
          pl.BlockSpec((_MLP_BLK, D), lambda i: (i, 0)),
          pl.BlockSpec((_MLP_BLK, D), lambda i: (i, 0)),
      ],
      out_shape=[
          jax.ShapeDtypeStruct((NP, D), jnp.float32),
          jax.ShapeDtypeStruct((NP, D), jnp.float32),
      ],
  )(xp, W1, b1, W2, b2, disb)


def kernel(x, edge_index, W1, b1, W2, b2):
  src = edge_index[0].astype(jnp.int32)
  dst = edge_index[1].astype(jnp.int32)
  xp = jnp.pad(x, ((0, NP - N), (0, 0)))
  srcl, dstl, cnt, disb = _pass_a(src, dst)
  h, outp = _mlp(xp, W1, b1.reshape(1, -1), W2, b2.reshape(1, -1), disb)
  out = h
  for _ in range(K):
    outp, out = _step(outp, h, disb, srcl, dstl, cnt)
  return out[:N]
